# Initial kernel scaffold; baseline (speedup 1.0000x reference)
import jax
import jax.numpy as jnp
from jax.experimental import pallas as pl


def kernel(h, edge_index):
    raise NotImplementedError("write your pallas kernel here")


# speedup vs baseline: 1.1005x; 1.1005x over previous
import functools

import jax
import jax.numpy as jnp
from jax import lax
from jax.experimental import pallas as pl
from jax.experimental.pallas import tpu as pltpu
from jax.experimental.pallas import tpu_sc as plsc

D = 128
L = 16
NC = 2
NS = 16
NW = NC * NS


def _dot_body(h_hbm, src_hbm, dst_hbm, out_hbm,
              idx_u, idx_v, rows_u, rows_v, scores_v, sem_u, sem_v,
              *, epw, chunk):
    wid = lax.axis_index("s") * NC + lax.axis_index("c")
    base = wid * epw
    nchunks = epw // chunk
    ngroups = chunk // L

    def chunk_body(c, carry):
        off = base + c * chunk
        pltpu.sync_copy(src_hbm.at[pl.ds(off, chunk)], idx_u)
        pltpu.sync_copy(dst_hbm.at[pl.ds(off, chunk)], idx_v)
        cp_u = pltpu.async_copy(h_hbm.at[idx_u], rows_u, sem_u)
        cp_v = pltpu.async_copy(h_hbm.at[idx_v], rows_v, sem_v)
        cp_u.wait()
        cp_v.wait()

        def group_body(g, gcarry):
            row_ids = g * L + lax.iota(jnp.int32, L)

            def d_body(dd, acc):
                col = jnp.full((L,), dd, jnp.int32)
                u = plsc.load_gather(rows_u, [row_ids, col])
                v = plsc.load_gather(rows_v, [row_ids, col])
                return acc + u * v

            acc = lax.fori_loop(0, D, d_body, jnp.zeros((L,), jnp.float32),
                                unroll=8)
            scores_v[pl.ds(g * L, L)] = acc
            return gcarry

        lax.fori_loop(0, ngroups, group_body, 0)
        pltpu.sync_copy(scores_v, out_hbm.at[pl.ds(off, chunk)])
        return carry

    lax.fori_loop(0, nchunks, chunk_body, 0)


def kernel(h, edge_index):
    E = edge_index.shape[1]
    epw = E // NW
    chunk = 80
    src = edge_index[0].astype(jnp.int32)
    dst = edge_index[1].astype(jnp.int32)
    mesh = plsc.VectorSubcoreMesh(core_axis_name="c", subcore_axis_name="s")
    body = functools.partial(_dot_body, epw=epw, chunk=chunk)
    f = pl.kernel(
        body,
        mesh=mesh,
        compiler_params=pltpu.CompilerParams(needs_layout_passes=False),
        out_type=jax.ShapeDtypeStruct((E,), jnp.float32),
        scratch_types=[
            pltpu.VMEM((chunk,), jnp.int32),
            pltpu.VMEM((chunk,), jnp.int32),
            pltpu.VMEM((chunk, D), jnp.float32),
            pltpu.VMEM((chunk, D), jnp.float32),
            pltpu.VMEM((chunk,), jnp.float32),
            pltpu.SemaphoreType.DMA,
            pltpu.SemaphoreType.DMA,
        ],
    )
    return f(h, src, dst)


# speedup vs baseline: 1.2301x; 1.1177x over previous
import functools

import jax
import jax.numpy as jnp
from jax import lax
from jax.experimental import pallas as pl
from jax.experimental.pallas import tpu as pltpu
from jax.experimental.pallas import tpu_sc as plsc

D = 128
L = 16
NC = 2
NS = 16
NW = NC * NS


def _dot_body(h_hbm, src_hbm, dst_hbm, out_hbm,
              h_sp, idx_u, idx_v, rows_u, rows_v, scores_c, sems,
              *, epw, chunk, n_nodes):
    sid = lax.axis_index("s")
    wid = sid * NC + lax.axis_index("c")
    base = wid * epw
    nchunks = epw // chunk
    ngroups = chunk // L

    rows_per_tile = (n_nodes // NS) // 8 * 8
    tail = n_nodes - rows_per_tile * NS
    pltpu.sync_copy(h_hbm.at[pl.ds(sid * rows_per_tile, rows_per_tile)],
                    h_sp.at[pl.ds(sid * rows_per_tile, rows_per_tile)])
    if tail:
        @pl.when(sid == NS - 1)
        def _():
            pltpu.sync_copy(h_hbm.at[pl.ds(rows_per_tile * NS, tail)],
                            h_sp.at[pl.ds(rows_per_tile * NS, tail)])
    plsc.subcore_barrier()

    def load_idx(c, slot):
        off = base + c * chunk
        pltpu.sync_copy(src_hbm.at[pl.ds(off, chunk)], idx_u.at[slot])
        pltpu.sync_copy(dst_hbm.at[pl.ds(off, chunk)], idx_v.at[slot])

    def issue(slot):
        pltpu.async_copy(h_sp.at[idx_u.at[slot]], rows_u.at[slot],
                         sems.at[slot, 0])
        pltpu.async_copy(h_sp.at[idx_v.at[slot]], rows_v.at[slot],
                         sems.at[slot, 1])

    def wait(slot):
        pltpu.make_async_copy(h_sp.at[idx_u.at[slot]], rows_u.at[slot],
                              sems.at[slot, 0]).wait()
        pltpu.make_async_copy(h_sp.at[idx_v.at[slot]], rows_v.at[slot],
                              sems.at[slot, 1]).wait()

    def compute(slot):
        ru = rows_u.at[slot]
        rv = rows_v.at[slot]

        def group_body(g, gcarry):
            row_ids = g * L + lax.iota(jnp.int32, L)

            def d_body(dd, acc):
                col = jnp.full((L,), dd, jnp.int32)
                u = plsc.load_gather(ru, [row_ids, col])
                v = plsc.load_gather(rv, [row_ids, col])
                return acc + u * v

            acc = lax.fori_loop(0, D, d_body, jnp.zeros((L,), jnp.float32),
                                unroll=8)
            scores_c[pl.ds(g * L, L)] = acc
            return gcarry

        lax.fori_loop(0, ngroups, group_body, 0)

    load_idx(0, 0)
    issue(0)

    def chunk_body(c, carry):
        def do(slot, other):
            wait(slot)

            @pl.when(c + 1 < nchunks)
            def _():
                load_idx(c + 1, other)
                issue(other)

            compute(slot)
            pltpu.sync_copy(scores_c,
                            out_hbm.at[pl.ds(base + c * chunk, chunk)])

        @pl.when(c % 2 == 0)
        def _():
            do(0, 1)

        @pl.when(c % 2 == 1)
        def _():
            do(1, 0)

        return carry

    lax.fori_loop(0, nchunks, chunk_body, 0)


def kernel(h, edge_index):
    E = edge_index.shape[1]
    epw = E // NW
    chunk = 80
    n_nodes = h.shape[0]
    src = edge_index[0].astype(jnp.int32)
    dst = edge_index[1].astype(jnp.int32)
    mesh = plsc.VectorSubcoreMesh(core_axis_name="c", subcore_axis_name="s")
    body = functools.partial(_dot_body, epw=epw, chunk=chunk,
                             n_nodes=n_nodes)
    f = pl.kernel(
        body,
        mesh=mesh,
        compiler_params=pltpu.CompilerParams(needs_layout_passes=False),
        out_type=jax.ShapeDtypeStruct((E,), jnp.float32),
        scratch_types=[
            pltpu.VMEM_SHARED((n_nodes, D), jnp.float32),
            pltpu.VMEM((2, chunk), jnp.int32),
            pltpu.VMEM((2, chunk), jnp.int32),
            pltpu.VMEM((2, chunk, D), jnp.float32),
            pltpu.VMEM((2, chunk, D), jnp.float32),
            pltpu.VMEM((chunk,), jnp.float32),
            pltpu.SemaphoreType.DMA((2, 2)),
        ],
    )
    return f(h, src, dst)


# speedup vs baseline: 7.8893x; 6.4137x over previous
import functools

import jax
import jax.numpy as jnp
from jax import lax
from jax.experimental import pallas as pl
from jax.experimental.pallas import tpu as pltpu
from jax.experimental.pallas import tpu_sc as plsc

D = 128
W = D // 2
L = 16
NC = 2
NS = 16
NW = NC * NS
NBUF = 4


def _dot_body(hp_hbm, cidx_hbm, out_hbm,
              h_sp, cidx, rows, scores2, tr, gsems, osems,
              *, epw, chunk, n_nodes):
    sid = lax.axis_index("s")
    wid = sid * NC + lax.axis_index("c")
    base = wid * epw
    nchunks = epw // chunk
    ngroups = chunk // L
    cw = 2 * chunk

    rows_per_tile = (n_nodes // NS) // 8 * 8
    tail = n_nodes - rows_per_tile * NS
    pltpu.sync_copy(hp_hbm.at[pl.ds(sid * rows_per_tile, rows_per_tile)],
                    h_sp.at[pl.ds(sid * rows_per_tile, rows_per_tile)])
    if tail:
        @pl.when(sid == NS - 1)
        def _():
            pltpu.sync_copy(hp_hbm.at[pl.ds(rows_per_tile * NS, tail)],
                            h_sp.at[pl.ds(rows_per_tile * NS, tail)])
    pltpu.sync_copy(cidx_hbm.at[pl.ds(wid * 2 * epw, 2 * epw)], cidx)
    plsc.subcore_barrier()

    def issue(c, slot):
        ic = cidx.at[pl.ds(c * cw, cw)]
        pltpu.async_copy(h_sp.at[ic], rows.at[slot], gsems.at[slot])

    def wait_gather(c, slot):
        ic = cidx.at[pl.ds(c * cw, cw)]
        pltpu.make_async_copy(h_sp.at[ic], rows.at[slot],
                              gsems.at[slot]).wait()

    col16 = lax.iota(jnp.int32, L) * L

    def compute(slot, sslot):
        rr = rows.at[slot]
        sc = scores2.at[sslot]

        def group_body(g, gcarry):
            def edge_body(e, ecarry):
                ea = g * L + e
                acc_lo = jnp.zeros((L,), jnp.float32)
                acc_hi = jnp.zeros((L,), jnp.float32)
                for k in range(W // L):
                    uw = rr[ea, pl.ds(k * L, L)]
                    vw = rr[chunk + ea, pl.ds(k * L, L)]
                    ub = plsc.bitcast(uw, jnp.bfloat16)
                    vb = plsc.bitcast(vw, jnp.bfloat16)
                    prod = ub * vb
                    pe, po = plsc.unpack(prod,
                                         format=plsc.PackFormat.INTERLEAVED)
                    acc_lo = acc_lo + pe
                    acc_hi = acc_hi + po
                acc = acc_lo + acc_hi
                plsc.store_scatter(tr, [col16 + e], acc)
                return ecarry

            lax.fori_loop(0, L, edge_body, 0, unroll=2)

            s = tr[pl.ds(0, L)]
            for i in range(1, L):
                s = s + tr[pl.ds(i * L, L)]
            sc[pl.ds(g * L, L)] = s
            return gcarry

        lax.fori_loop(0, ngroups, group_body, 0)

    def store_scores(c, sslot):
        pltpu.async_copy(scores2.at[sslot],
                         out_hbm.at[pl.ds(base + c * chunk, chunk)],
                         osems.at[sslot])

    def wait_store(c, sslot):
        pltpu.make_async_copy(scores2.at[sslot],
                              out_hbm.at[pl.ds(base + c * chunk, chunk)],
                              osems.at[sslot]).wait()

    for s in range(min(NBUF - 1, nchunks)):
        issue(s, s)

    def chunk_body(c, carry):
        def do(slot):
            wait_gather(c, slot)

            @pl.when(c + NBUF - 1 < nchunks)
            def _():
                issue(c + NBUF - 1, (slot + NBUF - 1) % NBUF)

            sslot = slot % 2

            @pl.when(c >= 2)
            def _():
                wait_store(c - 2, sslot)

            compute(slot, sslot)
            store_scores(c, sslot)

        for s in range(NBUF):
            @pl.when(c % NBUF == s)
            def _(s=s):
                do(s)

        return carry

    lax.fori_loop(0, nchunks, chunk_body, 0)
    wait_store(nchunks - 2, (nchunks - 2) % 2)
    wait_store(nchunks - 1, (nchunks - 1) % 2)


def kernel(h, edge_index):
    E = edge_index.shape[1]
    epw = E // NW
    chunk = 80
    nchunks = epw // chunk
    n_nodes = h.shape[0]
    hb = h.astype(jnp.bfloat16)
    hp = jax.lax.bitcast_convert_type(
        hb.reshape(n_nodes, W, 2), jnp.int32)
    src = edge_index[0].astype(jnp.int32).reshape(NW, nchunks, chunk)
    dst = edge_index[1].astype(jnp.int32).reshape(NW, nchunks, chunk)
    cidx = jnp.concatenate([src, dst], axis=-1).reshape(-1)
    mesh = plsc.VectorSubcoreMesh(core_axis_name="c", subcore_axis_name="s")
    body = functools.partial(_dot_body, epw=epw, chunk=chunk,
                             n_nodes=n_nodes)
    f = pl.kernel(
        body,
        mesh=mesh,
        compiler_params=pltpu.CompilerParams(needs_layout_passes=False,
                                             use_tc_tiling_on_sc=False),
        out_type=jax.ShapeDtypeStruct((E,), jnp.float32),
        scratch_types=[
            pltpu.VMEM_SHARED((n_nodes, W), jnp.int32),
            pltpu.VMEM((2 * epw,), jnp.int32),
            pltpu.VMEM((NBUF, 2 * chunk, W), jnp.int32),
            pltpu.VMEM((2, chunk), jnp.float32),
            pltpu.VMEM((L * L,), jnp.float32),
            pltpu.SemaphoreType.DMA((NBUF,)),
            pltpu.SemaphoreType.DMA((2,)),
        ],
    )
    return f(hp, cidx)


# speedup vs baseline: 11.1943x; 1.4189x over previous
import functools

import jax
import jax.numpy as jnp
from jax import lax
from jax.experimental import pallas as pl
from jax.experimental.pallas import tpu as pltpu
from jax.experimental.pallas import tpu_sc as plsc

D = 128
W = D // 2
L = 16
NC = 2
NS = 16
NW = NC * NS
NBUF = 4


def _dot_body(hp_hbm, cidx_hbm, out_hbm,
              h_sp, cidx, rows, scores2, tr, gsems, osems,
              *, epw, chunk, n_nodes):
    sid = lax.axis_index("s")
    wid = sid * NC + lax.axis_index("c")
    base = wid * epw
    nchunks = epw // chunk
    ngroups = chunk // L
    cw = 2 * chunk

    rows_per_tile = (n_nodes // NS) // 8 * 8
    tail = n_nodes - rows_per_tile * NS
    pltpu.sync_copy(hp_hbm.at[pl.ds(sid * rows_per_tile, rows_per_tile)],
                    h_sp.at[pl.ds(sid * rows_per_tile, rows_per_tile)])
    if tail:
        @pl.when(sid == NS - 1)
        def _():
            pltpu.sync_copy(hp_hbm.at[pl.ds(rows_per_tile * NS, tail)],
                            h_sp.at[pl.ds(rows_per_tile * NS, tail)])
    pltpu.sync_copy(cidx_hbm.at[pl.ds(wid * 2 * epw, 2 * epw)], cidx)
    plsc.subcore_barrier()

    def issue(c, slot):
        ic = cidx.at[pl.ds(c * cw, cw)]
        pltpu.async_copy(h_sp.at[ic], rows.at[slot], gsems.at[slot])

    def wait_gather(c, slot):
        ic = cidx.at[pl.ds(c * cw, cw)]
        pltpu.make_async_copy(h_sp.at[ic], rows.at[slot],
                              gsems.at[slot]).wait()

    col16 = lax.iota(jnp.int32, L) * L

    def compute(slot, sslot):
        rr = rows.at[slot]
        sc = scores2.at[sslot]

        def group_body(g, gcarry):
            @plsc.parallel_loop(0, L, step=1, unroll=4)
            def edge_body(e):
                ea = g * L + e
                acc_lo = None
                acc_hi = None
                for k in range(W // L):
                    uw = rr[ea, pl.ds(k * L, L)]
                    vw = rr[chunk + ea, pl.ds(k * L, L)]
                    ub = plsc.bitcast(uw, jnp.bfloat16)
                    vb = plsc.bitcast(vw, jnp.bfloat16)
                    prod = ub * vb
                    pe, po = plsc.unpack(prod,
                                         format=plsc.PackFormat.INTERLEAVED)
                    acc_lo = pe if acc_lo is None else acc_lo + pe
                    acc_hi = po if acc_hi is None else acc_hi + po
                acc = acc_lo + acc_hi
                plsc.store_scatter(tr, [col16 + e], acc)

            terms = [tr[pl.ds(i * L, L)] for i in range(L)]
            while len(terms) > 1:
                terms = [a + b for a, b in zip(terms[::2], terms[1::2])]
            sc[pl.ds(g * L, L)] = terms[0]
            return gcarry

        lax.fori_loop(0, ngroups, group_body, 0)

    def store_scores(c, sslot):
        pltpu.async_copy(scores2.at[sslot],
                         out_hbm.at[pl.ds(base + c * chunk, chunk)],
                         osems.at[sslot])

    def wait_store(c, sslot):
        pltpu.make_async_copy(scores2.at[sslot],
                              out_hbm.at[pl.ds(base + c * chunk, chunk)],
                              osems.at[sslot]).wait()

    for s in range(min(NBUF - 1, nchunks)):
        issue(s, s)

    def chunk_body(c, carry):
        def do(slot):
            wait_gather(c, slot)

            @pl.when(c + NBUF - 1 < nchunks)
            def _():
                issue(c + NBUF - 1, (slot + NBUF - 1) % NBUF)

            sslot = slot % 2

            @pl.when(c >= 2)
            def _():
                wait_store(c - 2, sslot)

            compute(slot, sslot)
            store_scores(c, sslot)

        for s in range(NBUF):
            @pl.when(c % NBUF == s)
            def _(s=s):
                do(s)

        return carry

    lax.fori_loop(0, nchunks, chunk_body, 0)
    wait_store(nchunks - 2, (nchunks - 2) % 2)
    wait_store(nchunks - 1, (nchunks - 1) % 2)


def kernel(h, edge_index):
    E = edge_index.shape[1]
    epw = E // NW
    chunk = 80
    nchunks = epw // chunk
    n_nodes = h.shape[0]
    hb = h.astype(jnp.bfloat16)
    hp = jax.lax.bitcast_convert_type(
        hb.reshape(n_nodes, W, 2), jnp.int32)
    src = edge_index[0].astype(jnp.int32).reshape(NW, nchunks, chunk)
    dst = edge_index[1].astype(jnp.int32).reshape(NW, nchunks, chunk)
    cidx = jnp.concatenate([src, dst], axis=-1).reshape(-1)
    mesh = plsc.VectorSubcoreMesh(core_axis_name="c", subcore_axis_name="s")
    body = functools.partial(_dot_body, epw=epw, chunk=chunk,
                             n_nodes=n_nodes)
    f = pl.kernel(
        body,
        mesh=mesh,
        compiler_params=pltpu.CompilerParams(needs_layout_passes=False,
                                             use_tc_tiling_on_sc=False),
        out_type=jax.ShapeDtypeStruct((E,), jnp.float32),
        scratch_types=[
            pltpu.VMEM_SHARED((n_nodes, W), jnp.int32),
            pltpu.VMEM((2 * epw,), jnp.int32),
            pltpu.VMEM((NBUF, 2 * chunk, W), jnp.int32),
            pltpu.VMEM((2, chunk), jnp.float32),
            pltpu.VMEM((L * L,), jnp.float32),
            pltpu.SemaphoreType.DMA((NBUF,)),
            pltpu.SemaphoreType.DMA((2,)),
        ],
    )
    return f(hp, cidx)
